# initial kernel scaffold (unmeasured)
import jax
import jax.numpy as jnp
from jax import lax
from jax.experimental import pallas as pl
from jax.experimental.pallas import tpu as pltpu

B, S, H, Dh, Dr = 4, 256, 32, 128, 64
D = 4096
DC_SH = 128
T = B * S
N_KV = H * Dh
SCALE = (Dh + Dr) ** -0.5

_BF = jnp.bfloat16
_F32 = jnp.float32


def _dot(a, b, dims):
    return lax.dot_general(a, b, (dims, ((), ())),
                           preferred_element_type=_F32)


def _kv_exchange_body(x_ref, wdkv_ref, wuk_ref, wuv_ref, wkr_ref,
                      k_ref, v_ref, kr_ref,
                      c_send, c_recv, wuk_send, wuk_recv,
                      wuv_send, wuv_recv, send_sems, recv_sems):
    my_x = lax.axis_index("x")
    my_y = lax.axis_index("y")
    my_z = lax.axis_index("z")
    peer = (1 - my_x, my_y, my_z)

    xv = x_ref[...]
    c = _dot(xv, wdkv_ref[...].astype(_BF), ((1,), (0,)))
    c_send[...] = c.astype(_BF)
    wuk_send[...] = wuk_ref[...].astype(_BF)
    wuv_send[...] = wuv_ref[...].astype(_BF)

    barrier = pltpu.get_barrier_semaphore()
    pl.semaphore_signal(barrier, inc=1, device_id=peer,
                        device_id_type=pl.DeviceIdType.MESH)
    pl.semaphore_wait(barrier, 1)

    rdmas = []
    for i, (src, dst) in enumerate([(c_send, c_recv),
                                    (wuk_send, wuk_recv),
                                    (wuv_send, wuv_recv)]):
        rdma = pltpu.make_async_remote_copy(
            src_ref=src, dst_ref=dst,
            send_sem=send_sems.at[i], recv_sem=recv_sems.at[i],
            device_id=peer, device_id_type=pl.DeviceIdType.MESH)
        rdma.start()
        rdmas.append(rdma)

    kr_ref[...] = _dot(xv, wkr_ref[...].astype(_BF),
                       ((1,), (0,))).astype(_BF)

    for rdma in rdmas:
        rdma.wait()

    c_own = c_send[...]
    c_peer = c_recv[...]
    nblk = 4
    bn = N_KV // nblk
    for j in range(nblk):
        sl = slice(j * bn, (j + 1) * bn)
        k_ref[:, sl] = (_dot(c_own, wuk_send[:, sl], ((1,), (0,)))
                        + _dot(c_peer, wuk_recv[:, sl],
                               ((1,), (0,)))).astype(_BF)
        v_ref[:, sl] = (_dot(c_own, wuv_send[:, sl], ((1,), (0,)))
                        + _dot(c_peer, wuv_recv[:, sl],
                               ((1,), (0,)))).astype(_BF)


def _kv_exchange(x_bf, Wdkv, Wuk, Wuv, Wkr):
    return pl.pallas_call(
        _kv_exchange_body,
        out_shape=[
            jax.ShapeDtypeStruct((T, N_KV), _BF),
            jax.ShapeDtypeStruct((T, N_KV), _BF),
            jax.ShapeDtypeStruct((T, Dr), _BF),
        ],
        in_specs=[pl.BlockSpec(memory_space=pltpu.VMEM)] * 5,
        out_specs=[pl.BlockSpec(memory_space=pltpu.VMEM)] * 3,
        scratch_shapes=[
            pltpu.VMEM((T, DC_SH), _BF),
            pltpu.VMEM((T, DC_SH), _BF),
            pltpu.VMEM((DC_SH, N_KV), _BF),
            pltpu.VMEM((DC_SH, N_KV), _BF),
            pltpu.VMEM((DC_SH, N_KV), _BF),
            pltpu.VMEM((DC_SH, N_KV), _BF),
            pltpu.SemaphoreType.DMA((3,)),
            pltpu.SemaphoreType.DMA((3,)),
        ],
        compiler_params=pltpu.CompilerParams(collective_id=0),
    )(x_bf, Wdkv, Wuk, Wuv, Wkr)


def _matmul(x_bf, w, out_dtype, nblk):
    t, din = x_bf.shape
    n = w.shape[1]
    bn = n // nblk

    def body(x_ref, w_ref, o_ref):
        o_ref[...] = _dot(x_ref[...], w_ref[...].astype(_BF),
                          ((1,), (0,))).astype(out_dtype)

    return pl.pallas_call(
        body,
        grid=(nblk,),
        in_specs=[
            pl.BlockSpec((t, din), lambda j: (0, 0)),
            pl.BlockSpec((din, bn), lambda j: (0, j)),
        ],
        out_specs=pl.BlockSpec((t, bn), lambda j: (0, j)),
        out_shape=jax.ShapeDtypeStruct((t, n), out_dtype),
    )(x_bf, w)


def _attention(Q, K, V, Qr, Kr):

    def body(q_ref, k_ref, v_ref, qr_ref, kr_ref, o_ref):
        q = q_ref[0]
        k = k_ref[0]
        v = v_ref[0]
        s = _dot(q, k, ((1,), (1,)))
        s += _dot(qr_ref[0], kr_ref[0], ((1,), (1,)))
        s *= SCALE
        m = jnp.max(s, axis=1, keepdims=True)
        p = jnp.exp(s - m)
        p = p / jnp.sum(p, axis=1, keepdims=True)
        o = _dot(p.astype(_BF), v, ((1,), (0,)))
        o_ref[0] = o.astype(_BF)

    return pl.pallas_call(
        body,
        grid=(B, H),
        in_specs=[
            pl.BlockSpec((1, S, Dh), lambda b, h: (b, 0, h)),
            pl.BlockSpec((1, S, Dh), lambda b, h: (b, 0, h)),
            pl.BlockSpec((1, S, Dh), lambda b, h: (b, 0, h)),
            pl.BlockSpec((1, S, Dr), lambda b, h: (b, 0, h)),
            pl.BlockSpec((1, S, Dr), lambda b, h: (b, 0, 0)),
        ],
        out_specs=pl.BlockSpec((1, S, Dh), lambda b, h: (b, 0, h)),
        out_shape=jax.ShapeDtypeStruct((B, S, N_KV), _BF),
    )(Q, K, V, Qr, Kr)


def kernel(x, Wdkv, Wuk, Wuv, Wq, Wqr, Wkr, Wo):
    x_bf = x.reshape(T, D).astype(_BF)

    K, V, Kr = _kv_exchange(x_bf, Wdkv, Wuk, Wuv, Wkr)
    Q = _matmul(x_bf, Wq, _BF, 8)
    Qr = _matmul(x_bf, Wqr, _BF, 4)

    O = _attention(
        Q.reshape(B, S, N_KV),
        K.reshape(B, S, N_KV),
        V.reshape(B, S, N_KV),
        Qr.reshape(B, S, H * Dr),
        Kr.reshape(B, S, Dr),
    )

    out = _matmul(O.reshape(T, N_KV), Wo, _F32, 8)
    return out.reshape(B, S, D)


# baseline (device time: 246925 ns/iter reference)
import jax
import jax.numpy as jnp
from jax import lax
from jax.experimental import pallas as pl
from jax.experimental.pallas import tpu as pltpu

B, S, H, Dh, Dr = 4, 256, 32, 128, 64
D = 4096
DC_SH = 128
T = B * S
N_KV = H * Dh
SCALE = (Dh + Dr) ** -0.5

_BF = jnp.bfloat16
_F32 = jnp.float32


def _dot(a, b, dims):
    return lax.dot_general(a, b, (dims, ((), ())),
                           preferred_element_type=_F32)


def _kv_exchange_body(x_ref, wdkv_ref, wuk_ref, wuv_ref, wkr_ref,
                      k_ref, v_ref, kr_ref,
                      c_send, c_recv, wuk_send, wuk_recv,
                      wuv_send, wuv_recv, send_sems, recv_sems):
    my_x = lax.axis_index("x")
    my_y = lax.axis_index("y")
    my_z = lax.axis_index("z")
    peer = (1 - my_x, my_y, my_z)

    xv = x_ref[...]
    c = _dot(xv, wdkv_ref[...].astype(_BF), ((1,), (0,)))
    c_send[...] = c.astype(_BF)
    wuk_send[...] = wuk_ref[...].astype(_BF)
    wuv_send[...] = wuv_ref[...].astype(_BF)

    barrier = pltpu.get_barrier_semaphore()
    pl.semaphore_signal(barrier, inc=1, device_id=peer,
                        device_id_type=pl.DeviceIdType.MESH)
    pl.semaphore_wait(barrier, 1)

    rdmas = []
    for i, (src, dst) in enumerate([(c_send, c_recv),
                                    (wuk_send, wuk_recv),
                                    (wuv_send, wuv_recv)]):
        rdma = pltpu.make_async_remote_copy(
            src_ref=src, dst_ref=dst,
            send_sem=send_sems.at[i], recv_sem=recv_sems.at[i],
            device_id=peer, device_id_type=pl.DeviceIdType.MESH)
        rdma.start()
        rdmas.append(rdma)

    kr_ref[...] = _dot(xv, wkr_ref[...].astype(_BF),
                       ((1,), (0,))).astype(_BF)

    for rdma in rdmas:
        rdma.wait()

    c_own = c_send[...]
    c_peer = c_recv[...]
    nblk = 4
    bn = N_KV // nblk
    for j in range(nblk):
        sl = slice(j * bn, (j + 1) * bn)
        k_ref[:, sl] = (_dot(c_own, wuk_send[:, sl], ((1,), (0,)))
                        + _dot(c_peer, wuk_recv[:, sl],
                               ((1,), (0,)))).astype(_BF)
        v_ref[:, sl] = (_dot(c_own, wuv_send[:, sl], ((1,), (0,)))
                        + _dot(c_peer, wuv_recv[:, sl],
                               ((1,), (0,)))).astype(_BF)


def _kv_exchange(x_bf, Wdkv, Wuk, Wuv, Wkr):
    return pl.pallas_call(
        _kv_exchange_body,
        out_shape=[
            jax.ShapeDtypeStruct((T, N_KV), _BF),
            jax.ShapeDtypeStruct((T, N_KV), _BF),
            jax.ShapeDtypeStruct((T, Dr), _BF),
        ],
        in_specs=[pl.BlockSpec(memory_space=pltpu.VMEM)] * 5,
        out_specs=[pl.BlockSpec(memory_space=pltpu.VMEM)] * 3,
        scratch_shapes=[
            pltpu.VMEM((T, DC_SH), _BF),
            pltpu.VMEM((T, DC_SH), _BF),
            pltpu.VMEM((DC_SH, N_KV), _BF),
            pltpu.VMEM((DC_SH, N_KV), _BF),
            pltpu.VMEM((DC_SH, N_KV), _BF),
            pltpu.VMEM((DC_SH, N_KV), _BF),
            pltpu.SemaphoreType.DMA((3,)),
            pltpu.SemaphoreType.DMA((3,)),
        ],
        compiler_params=pltpu.CompilerParams(collective_id=0),
    )(x_bf, Wdkv, Wuk, Wuv, Wkr)


def _matmul(x_bf, w, out_dtype, nblk):
    t, din = x_bf.shape
    n = w.shape[1]
    bn = n // nblk

    def body(x_ref, w_ref, o_ref):
        o_ref[...] = _dot(x_ref[...], w_ref[...].astype(_BF),
                          ((1,), (0,))).astype(out_dtype)

    return pl.pallas_call(
        body,
        grid=(nblk,),
        in_specs=[
            pl.BlockSpec((t, din), lambda j: (0, 0)),
            pl.BlockSpec((din, bn), lambda j: (0, j)),
        ],
        out_specs=pl.BlockSpec((t, bn), lambda j: (0, j)),
        out_shape=jax.ShapeDtypeStruct((t, n), out_dtype),
    )(x_bf, w)


def _attention(Q, K, V, Qr, Kr):
    HP = 2

    def body(q_ref, k_ref, v_ref, qr_ref, kr_ref, o_ref):
        kr = kr_ref[0]
        for i in range(HP):
            hs = slice(i * Dh, (i + 1) * Dh)
            rs = slice(i * Dr, (i + 1) * Dr)
            s = _dot(q_ref[0][:, hs], k_ref[0][:, hs], ((1,), (1,)))
            s += _dot(qr_ref[0][:, rs], kr, ((1,), (1,)))
            s *= SCALE
            m = jnp.max(s, axis=1, keepdims=True)
            p = jnp.exp(s - m)
            p = p / jnp.sum(p, axis=1, keepdims=True)
            o = _dot(p.astype(_BF), v_ref[0][:, hs], ((1,), (0,)))
            o_ref[0, :, hs] = o.astype(_BF)

    return pl.pallas_call(
        body,
        grid=(B, H // HP),
        in_specs=[
            pl.BlockSpec((1, S, HP * Dh), lambda b, h: (b, 0, h)),
            pl.BlockSpec((1, S, HP * Dh), lambda b, h: (b, 0, h)),
            pl.BlockSpec((1, S, HP * Dh), lambda b, h: (b, 0, h)),
            pl.BlockSpec((1, S, HP * Dr), lambda b, h: (b, 0, h)),
            pl.BlockSpec((1, S, Dr), lambda b, h: (b, 0, 0)),
        ],
        out_specs=pl.BlockSpec((1, S, HP * Dh), lambda b, h: (b, 0, h)),
        out_shape=jax.ShapeDtypeStruct((B, S, N_KV), _BF),
    )(Q, K, V, Qr, Kr)


def kernel(x, Wdkv, Wuk, Wuv, Wq, Wqr, Wkr, Wo):
    x_bf = x.reshape(T, D).astype(_BF)

    K, V, Kr = _kv_exchange(x_bf, Wdkv, Wuk, Wuv, Wkr)
    Q = _matmul(x_bf, Wq, _BF, 8)
    Qr = _matmul(x_bf, Wqr, _BF, 4)

    O = _attention(
        Q.reshape(B, S, N_KV),
        K.reshape(B, S, N_KV),
        V.reshape(B, S, N_KV),
        Qr.reshape(B, S, H * Dr),
        Kr.reshape(B, S, Dr),
    )

    out = _matmul(O.reshape(T, N_KV), Wo, _F32, 8)
    return out.reshape(B, S, D)


# device time: 189215 ns/iter; 1.3050x vs baseline; 1.3050x over previous
import jax
import jax.numpy as jnp
from jax import lax
from jax.experimental import pallas as pl
from jax.experimental.pallas import tpu as pltpu

B, S, H, Dh, Dr = 4, 256, 32, 128, 64
D = 4096
DC_SH = 128
T = B * S
N_KV = H * Dh
SCALE = (Dh + Dr) ** -0.5

_BF = jnp.bfloat16
_F32 = jnp.float32

NBLK_Q = 16


def _dot(a, b, dims):
    return lax.dot_general(a, b, (dims, ((), ())),
                           preferred_element_type=_F32)


def _qkv_body(x_ref, wdkv_ref, wuk_ref, wuv_ref, wkr_ref, wq_ref,
              q_ref, k_ref, v_ref, kr_ref,
              c_send, c_recv, wuk_send, wuk_recv,
              wuv_send, wuv_recv, send_sems, recv_sems):
    j = pl.program_id(0)
    my_x = lax.axis_index("x")
    my_y = lax.axis_index("y")
    my_z = lax.axis_index("z")
    peer = (1 - my_x, my_y, my_z)

    def rdma(i, src, dst):
        return pltpu.make_async_remote_copy(
            src_ref=src, dst_ref=dst,
            send_sem=send_sems.at[i], recv_sem=recv_sems.at[i],
            device_id=peer, device_id_type=pl.DeviceIdType.MESH)

    pairs = [(c_send, c_recv), (wuk_send, wuk_recv), (wuv_send, wuv_recv)]

    @pl.when(j == 0)
    def _start():
        xv = x_ref[...]
        c = _dot(xv, wdkv_ref[...].astype(_BF), ((1,), (0,)))
        c_send[...] = c.astype(_BF)
        wuk_send[...] = wuk_ref[...].astype(_BF)
        wuv_send[...] = wuv_ref[...].astype(_BF)

        barrier = pltpu.get_barrier_semaphore()
        pl.semaphore_signal(barrier, inc=1, device_id=peer,
                            device_id_type=pl.DeviceIdType.MESH)
        pl.semaphore_wait(barrier, 1)

        for i, (src, dst) in enumerate(pairs):
            rdma(i, src, dst).start()

        kr_ref[...] = _dot(xv, wkr_ref[...].astype(_BF),
                           ((1,), (0,))).astype(_BF)

    q_ref[...] = _dot(x_ref[...], wq_ref[...].astype(_BF),
                      ((1,), (0,))).astype(_BF)

    @pl.when(j == NBLK_Q - 1)
    def _finish():
        for i, (src, dst) in enumerate(pairs):
            rdma(i, src, dst).wait()
        c_cat = jnp.concatenate([c_send[...], c_recv[...]], axis=1)
        wuk_cat = jnp.concatenate([wuk_send[...], wuk_recv[...]], axis=0)
        wuv_cat = jnp.concatenate([wuv_send[...], wuv_recv[...]], axis=0)
        nblk = 8
        bn = N_KV // nblk
        for jj in range(nblk):
            sl = slice(jj * bn, (jj + 1) * bn)
            k_ref[:, sl] = _dot(c_cat, wuk_cat[:, sl],
                                ((1,), (0,))).astype(_BF)
            v_ref[:, sl] = _dot(c_cat, wuv_cat[:, sl],
                                ((1,), (0,))).astype(_BF)


def _qkv(x_bf, Wdkv, Wuk, Wuv, Wkr, Wq):
    bn = D // NBLK_Q
    return pl.pallas_call(
        _qkv_body,
        grid=(NBLK_Q,),
        out_shape=[
            jax.ShapeDtypeStruct((T, D), _BF),
            jax.ShapeDtypeStruct((T, N_KV), _BF),
            jax.ShapeDtypeStruct((T, N_KV), _BF),
            jax.ShapeDtypeStruct((T, Dr), _BF),
        ],
        in_specs=[
            pl.BlockSpec((T, D), lambda j: (0, 0)),
            pl.BlockSpec((D, DC_SH), lambda j: (0, 0)),
            pl.BlockSpec((DC_SH, N_KV), lambda j: (0, 0)),
            pl.BlockSpec((DC_SH, N_KV), lambda j: (0, 0)),
            pl.BlockSpec((D, Dr), lambda j: (0, 0)),
            pl.BlockSpec((D, bn), lambda j: (0, j)),
        ],
        out_specs=[
            pl.BlockSpec((T, bn), lambda j: (0, j)),
            pl.BlockSpec((T, N_KV), lambda j: (0, 0)),
            pl.BlockSpec((T, N_KV), lambda j: (0, 0)),
            pl.BlockSpec((T, Dr), lambda j: (0, 0)),
        ],
        scratch_shapes=[
            pltpu.VMEM((T, DC_SH), _BF),
            pltpu.VMEM((T, DC_SH), _BF),
            pltpu.VMEM((DC_SH, N_KV), _BF),
            pltpu.VMEM((DC_SH, N_KV), _BF),
            pltpu.VMEM((DC_SH, N_KV), _BF),
            pltpu.VMEM((DC_SH, N_KV), _BF),
            pltpu.SemaphoreType.DMA((3,)),
            pltpu.SemaphoreType.DMA((3,)),
        ],
        compiler_params=pltpu.CompilerParams(
            collective_id=0, vmem_limit_bytes=100 * 1024 * 1024),
    )(x_bf, Wdkv, Wuk, Wuv, Wkr, Wq)


def _matmul(x_bf, w, out_dtype, nblk):
    t, din = x_bf.shape
    n = w.shape[1]
    bn = n // nblk

    def body(x_ref, w_ref, o_ref):
        o_ref[...] = _dot(x_ref[...], w_ref[...].astype(_BF),
                          ((1,), (0,))).astype(out_dtype)

    return pl.pallas_call(
        body,
        grid=(nblk,),
        in_specs=[
            pl.BlockSpec((t, din), lambda j: (0, 0)),
            pl.BlockSpec((din, bn), lambda j: (0, j)),
        ],
        out_specs=pl.BlockSpec((t, bn), lambda j: (0, j)),
        out_shape=jax.ShapeDtypeStruct((t, n), out_dtype),
    )(x_bf, w)


def _attention(Q, K, V, Qr, Kr):
    HP = 8

    def body(q_ref, k_ref, v_ref, qr_ref, kr_ref, o_ref):
        kr = kr_ref[0]
        for i in range(HP):
            hs = slice(i * Dh, (i + 1) * Dh)
            rs = slice(i * Dr, (i + 1) * Dr)
            q = q_ref[0][:, hs] * SCALE
            qr = qr_ref[0][:, rs] * SCALE
            s = _dot(q, k_ref[0][:, hs], ((1,), (1,)))
            s += _dot(qr, kr, ((1,), (1,)))
            p = jnp.exp(s)
            denom = jnp.sum(p, axis=1, keepdims=True)
            o = _dot(p.astype(_BF), v_ref[0][:, hs], ((1,), (0,)))
            o_ref[0, :, hs] = (o * (1.0 / denom)).astype(_BF)

    return pl.pallas_call(
        body,
        grid=(B, H // HP),
        in_specs=[
            pl.BlockSpec((1, S, HP * Dh), lambda b, h: (b, 0, h)),
            pl.BlockSpec((1, S, HP * Dh), lambda b, h: (b, 0, h)),
            pl.BlockSpec((1, S, HP * Dh), lambda b, h: (b, 0, h)),
            pl.BlockSpec((1, S, HP * Dr), lambda b, h: (b, 0, h)),
            pl.BlockSpec((1, S, Dr), lambda b, h: (b, 0, 0)),
        ],
        out_specs=pl.BlockSpec((1, S, HP * Dh), lambda b, h: (b, 0, h)),
        out_shape=jax.ShapeDtypeStruct((B, S, N_KV), _BF),
    )(Q, K, V, Qr, Kr)


def kernel(x, Wdkv, Wuk, Wuv, Wq, Wqr, Wkr, Wo):
    x_bf = x.reshape(T, D).astype(_BF)

    Q, K, V, Kr = _qkv(x_bf, Wdkv, Wuk, Wuv, Wkr, Wq)
    Qr = _matmul(x_bf, Wqr, _BF, 4)

    O = _attention(
        Q.reshape(B, S, N_KV),
        K.reshape(B, S, N_KV),
        V.reshape(B, S, N_KV),
        Qr.reshape(B, S, H * Dr),
        Kr.reshape(B, S, Dr),
    )

    out = _matmul(O.reshape(T, N_KV), Wo, _F32, 8)
    return out.reshape(B, S, D)
